# baseline (speedup 1.0000x reference)
import functools

import jax
import jax.numpy as jnp
from jax import lax
from jax.experimental import pallas as pl
from jax.experimental.pallas import tpu as pltpu
import jax.experimental.pallas.tpu_sc as plsc

N = 10000
E = 160000
SUB = 1000
G = 64
FEAT = 16
CS = 5

NPAD = 10240
NC = 2
NS = 16
NW = NC * NS
EPT = E // NW
CH = 1000
NCHUNK = EPT // CH
RPT = NPAD // NS


def _elu(v):
    return jnp.where(v > 0, v, jnp.exp(v) - 1.0)


def _sc_gather(table, idx, d):
    mesh = plsc.VectorSubcoreMesh(core_axis_name="c", subcore_axis_name="s")

    @functools.partial(
        pl.kernel,
        out_type=jax.ShapeDtypeStruct((E, d), jnp.float32),
        mesh=mesh,
        scratch_types=[
            pltpu.VMEM((CH,), jnp.int32),
            pltpu.VMEM((CH, d), jnp.float32),
            pltpu.SemaphoreType.DMA,
        ],
        compiler_params=pltpu.CompilerParams(use_tc_tiling_on_sc=False),
    )
    def k(table_hbm, idx_hbm, out_hbm, idx_v, rows_v, sem):
        wid = lax.axis_index("s") * NC + lax.axis_index("c")
        base = wid * EPT
        for j in range(NCHUNK):
            off = base + j * CH
            pltpu.sync_copy(idx_hbm.at[pl.ds(off, CH)], idx_v)
            pltpu.async_copy(table_hbm.at[idx_v], rows_v, sem).wait()
            pltpu.sync_copy(rows_v, out_hbm.at[pl.ds(off, CH)])

    return k(table, idx)


def _sc_scatter_add(msg, dst, zeros, d):
    mesh = plsc.VectorSubcoreMesh(core_axis_name="c", subcore_axis_name="s")

    @functools.partial(
        pl.kernel,
        out_type=jax.ShapeDtypeStruct((NC, NPAD, d), jnp.float32),
        mesh=mesh,
        scratch_types=[
            pltpu.VMEM((CH,), jnp.int32),
            pltpu.VMEM((CH, d), jnp.float32),
            pltpu.VMEM_SHARED((NPAD, d), jnp.float32),
            pltpu.SemaphoreType.DMA,
        ],
        compiler_params=pltpu.CompilerParams(use_tc_tiling_on_sc=False),
    )
    def k(msg_hbm, dst_hbm, z_hbm, out_hbm, idx_v, rows_v, acc, sem):
        c = lax.axis_index("c")
        s = lax.axis_index("s")
        rbase = s * RPT
        pltpu.sync_copy(z_hbm.at[pl.ds(rbase, RPT)], acc.at[pl.ds(rbase, RPT)])
        plsc.subcore_barrier()
        base = (s * NC + c) * EPT
        for j in range(NCHUNK):
            off = base + j * CH
            pltpu.sync_copy(dst_hbm.at[pl.ds(off, CH)], idx_v)
            pltpu.sync_copy(msg_hbm.at[pl.ds(off, CH)], rows_v)
            pltpu.sync_copy(rows_v, acc.at[idx_v], add=True)
        plsc.subcore_barrier()
        pltpu.sync_copy(acc.at[pl.ds(rbase, RPT)], out_hbm.at[c, pl.ds(rbase, RPT)])

    return k(msg, dst, zeros)


def _msg_body(m_in, m_out, ea_ref, xj_ref, w1_ref, b1_ref, w2_ref, b2_ref, o_ref):
    h = jnp.maximum(jnp.dot(ea_ref[...], w1_ref[...],
                            preferred_element_type=jnp.float32) + b1_ref[...], 0.0)
    we = jnp.dot(h.astype(jnp.bfloat16), w2_ref[...],
                 preferred_element_type=jnp.float32) + b2_ref[...]
    xj = xj_ref[...]
    acc = xj[:, 0:1] * we[:, 0:m_out]
    for i in range(1, m_in):
        acc = acc + xj[:, i:i + 1] * we[:, i * m_out:(i + 1) * m_out]
    o_ref[...] = acc


def _tc_msg(ea, xj, w1, b1, w2, b2, m_in, m_out, blk):
    grid = E // blk
    b1 = b1.reshape(1, -1)
    b2 = b2.reshape(1, -1)
    w2 = w2.astype(jnp.bfloat16)
    return pl.pallas_call(
        functools.partial(_msg_body, m_in, m_out),
        grid=(grid,),
        in_specs=[
            pl.BlockSpec((blk, 5), lambda i: (i, 0)),
            pl.BlockSpec((blk, xj.shape[1]), lambda i: (i, 0)),
            pl.BlockSpec(w1.shape, lambda i: (0, 0)),
            pl.BlockSpec(b1.shape, lambda i: (0, 0)),
            pl.BlockSpec(w2.shape, lambda i: (0, 0)),
            pl.BlockSpec(b2.shape, lambda i: (0, 0)),
        ],
        out_specs=pl.BlockSpec((blk, m_out), lambda i: (i, 0)),
        out_shape=jax.ShapeDtypeStruct((E, m_out), jnp.float32),
    )(ea, xj, w1, b1, w2, b2)


def _h1_body(agg_ref, x5_ref, root_ref, bias_ref, o_ref):
    a = agg_ref[0] + agg_ref[1]
    v = a + jnp.dot(x5_ref[...], root_ref[...],
                    preferred_element_type=jnp.float32) + bias_ref[...]
    o_ref[...] = _elu(v)


def _tc_h1(agg, x5, root, bias):
    blk = 1024
    grid = NPAD // blk
    return pl.pallas_call(
        _h1_body,
        grid=(grid,),
        in_specs=[
            pl.BlockSpec((NC, blk, 32), lambda i: (0, i, 0)),
            pl.BlockSpec((blk, 5), lambda i: (i, 0)),
            pl.BlockSpec((5, 32), lambda i: (0, 0)),
            pl.BlockSpec((1, 32), lambda i: (0, 0)),
        ],
        out_specs=pl.BlockSpec((blk, 32), lambda i: (i, 0)),
        out_shape=jax.ShapeDtypeStruct((NPAD, 32), jnp.float32),
    )(agg, x5, root, bias.reshape(1, 32))


def _pool1_body(agg_ref, h1_ref, root_ref, bias_ref, xc_ref, seg_ref, o_ref):
    i = pl.program_id(0)
    a = agg_ref[0] + agg_ref[1]
    h2 = _elu(a + jnp.dot(h1_ref[...], root_ref[...],
                          preferred_element_type=jnp.float32) + bias_ref[...])
    bn = h2.shape[0]
    ones = jnp.ones((bn, 1), jnp.float32)
    feat = jnp.concatenate([h2, xc_ref[...], ones], axis=1)
    seg = seg_ref[0]
    iota = lax.broadcasted_iota(jnp.int32, (SUB, bn), 0)
    oh = (iota == seg).astype(jnp.float32)

    @pl.when(i == 0)
    def _():
        o_ref[...] = jnp.zeros_like(o_ref)

    o_ref[...] += jnp.dot(oh, feat, preferred_element_type=jnp.float32)


def _tc_pool1(agg2, h1, root2, bias2, xc, seg3d):
    blk = 1024
    grid = NPAD // blk
    return pl.pallas_call(
        _pool1_body,
        grid=(grid,),
        in_specs=[
            pl.BlockSpec((NC, blk, 64), lambda i: (0, i, 0)),
            pl.BlockSpec((blk, 32), lambda i: (i, 0)),
            pl.BlockSpec((32, 64), lambda i: (0, 0)),
            pl.BlockSpec((1, 64), lambda i: (0, 0)),
            pl.BlockSpec((blk, 11), lambda i: (i, 0)),
            pl.BlockSpec((1, 1, blk), lambda i: (i, 0, 0)),
        ],
        out_specs=pl.BlockSpec((SUB, 76), lambda i: (0, 0)),
        out_shape=jax.ShapeDtypeStruct((SUB, 76), jnp.float32),
    )(agg2, h1, root2, bias2.reshape(1, 64), xc, seg3d)


def _final_body(ss_ref, s2g_ref, w1_ref, b1_ref, w2_ref, b2_ref, w3_ref, b3_ref,
                o_ref):
    ss = ss_ref[...]
    cnt1 = ss[:, 75:76]
    mean1 = ss / jnp.maximum(cnt1, 1.0)
    s2g = s2g_ref[...]
    iota = lax.broadcasted_iota(jnp.int32, (G, SUB), 0)
    oh = (iota == s2g).astype(jnp.float32)
    gsum = jnp.dot(oh, mean1, preferred_element_type=jnp.float32)
    cnt2 = jnp.sum(oh, axis=1, keepdims=True)
    gmean = gsum / jnp.maximum(cnt2, 1.0)
    h = gmean[:, :75]
    h = _elu(jnp.dot(h, w1_ref[...], preferred_element_type=jnp.float32) + b1_ref[...])
    h = _elu(jnp.dot(h, w2_ref[...], preferred_element_type=jnp.float32) + b2_ref[...])
    o_ref[...] = jnp.dot(h, w3_ref[...], preferred_element_type=jnp.float32) + b3_ref[...]


def _tc_final(sub_sums, s2g, fc1_W, fc1_b, fc2_W, fc2_b, fc3_W, fc3_b):
    return pl.pallas_call(
        _final_body,
        out_shape=jax.ShapeDtypeStruct((G, 1), jnp.float32),
    )(sub_sums, s2g.reshape(1, SUB), fc1_W, fc1_b.reshape(1, 32),
      fc2_W, fc2_b.reshape(1, 16), fc3_W, fc3_b.reshape(1, 1))


def kernel(x, edge_index, edge_attr, node_to_subgraph, subgraph_to_graph,
           nn1_W1, nn1_b1, nn1_W2, nn1_b2, root1, bias1,
           nn2_W1, nn2_b1, nn2_W2, nn2_b2, root2, bias2,
           fc1_W, fc1_b, fc2_W, fc2_b, fc3_W, fc3_b):
    src = edge_index[0]
    dst = edge_index[1]
    x_pad = jnp.pad(x, ((0, NPAD - N), (0, 0)))
    x5 = x_pad[:, :CS]
    xc = x_pad[:, CS:]

    xj1 = _sc_gather(x_pad, src, FEAT)
    msg1 = _tc_msg(edge_attr, xj1, nn1_W1, nn1_b1, nn1_W2, nn1_b2, CS, 32, 1600)
    agg1 = _sc_scatter_add(msg1, dst, jnp.zeros((NPAD, 32), jnp.float32), 32)
    h1 = _tc_h1(agg1, x5, root1, bias1)

    xj2 = _sc_gather(h1, src, 32)
    msg2 = _tc_msg(edge_attr, xj2, nn2_W1, nn2_b1, nn2_W2, nn2_b2, 32, 64, 1280)
    agg2 = _sc_scatter_add(msg2, dst, jnp.zeros((NPAD, 64), jnp.float32), 64)

    seg = jnp.concatenate(
        [node_to_subgraph, jnp.full((NPAD - N,), SUB, jnp.int32)]).reshape(
            NPAD // 1024, 1, 1024)
    sub_sums = _tc_pool1(agg2, h1, root2, bias2, xc, seg)
    out = _tc_final(sub_sums, subgraph_to_graph,
                    fc1_W, fc1_b, fc2_W, fc2_b, fc3_W, fc3_b)
    return out.reshape(-1)


# speedup vs baseline: 2.0544x; 2.0544x over previous
import functools

import jax
import jax.numpy as jnp
from jax import lax
from jax.experimental import pallas as pl
from jax.experimental.pallas import tpu as pltpu
import jax.experimental.pallas.tpu_sc as plsc

N = 10000
E = 160000
SUB = 1000
G = 64
FEAT = 16
CS = 5

NPAD = 10240
NC = 2
NS = 16
NW = NC * NS
EPT = E // NW
CH = 1000
NCHUNK = EPT // CH
RPT = NPAD // NS


def _elu(v):
    return jnp.where(v > 0, v, jnp.exp(v) - 1.0)


def _sc_gather(table, idx, d):
    mesh = plsc.VectorSubcoreMesh(core_axis_name="c", subcore_axis_name="s")

    @functools.partial(
        pl.kernel,
        out_type=jax.ShapeDtypeStruct((E, d), jnp.float32),
        mesh=mesh,
        scratch_types=[
            pltpu.VMEM((CH,), jnp.int32),
            pltpu.VMEM((CH, d), jnp.float32),
            pltpu.SemaphoreType.DMA,
        ],
        compiler_params=pltpu.CompilerParams(use_tc_tiling_on_sc=False),
    )
    def k(table_hbm, idx_hbm, out_hbm, idx_v, rows_v, sem):
        wid = lax.axis_index("s") * NC + lax.axis_index("c")
        base = wid * EPT
        for j in range(NCHUNK):
            off = base + j * CH
            pltpu.sync_copy(idx_hbm.at[pl.ds(off, CH)], idx_v)
            pltpu.async_copy(table_hbm.at[idx_v], rows_v, sem).wait()
            pltpu.sync_copy(rows_v, out_hbm.at[pl.ds(off, CH)])

    return k(table, idx)


def _sc_scatter_add(msg, dst, zeros, d):
    mesh = plsc.VectorSubcoreMesh(core_axis_name="c", subcore_axis_name="s")

    @functools.partial(
        pl.kernel,
        out_type=jax.ShapeDtypeStruct((NC, NPAD, d), jnp.float32),
        mesh=mesh,
        scratch_types=[
            pltpu.VMEM((CH,), jnp.int32),
            pltpu.VMEM((CH, d), jnp.float32),
            pltpu.VMEM_SHARED((NPAD, d), jnp.float32),
            pltpu.SemaphoreType.DMA,
        ],
        compiler_params=pltpu.CompilerParams(use_tc_tiling_on_sc=False),
    )
    def k(msg_hbm, dst_hbm, z_hbm, out_hbm, idx_v, rows_v, acc, sem):
        c = lax.axis_index("c")
        s = lax.axis_index("s")
        rbase = s * RPT
        pltpu.sync_copy(z_hbm.at[pl.ds(rbase, RPT)], acc.at[pl.ds(rbase, RPT)])
        plsc.subcore_barrier()
        base = (s * NC + c) * EPT
        for j in range(NCHUNK):
            off = base + j * CH
            pltpu.sync_copy(dst_hbm.at[pl.ds(off, CH)], idx_v)
            pltpu.sync_copy(msg_hbm.at[pl.ds(off, CH)], rows_v)
            pltpu.sync_copy(rows_v, acc.at[idx_v], add=True)
        plsc.subcore_barrier()
        pltpu.sync_copy(acc.at[pl.ds(rbase, RPT)], out_hbm.at[c, pl.ds(rbase, RPT)])

    return k(msg, dst, zeros)


def _msg_body(m_in, m_out, eat_ref, xj_ref, w1t_ref, w2t_ref, b2t_ref, o_ref):
    h_t = jnp.maximum(jnp.dot(w1t_ref[...], eat_ref[...],
                              preferred_element_type=jnp.float32), 0.0)
    we_t = jnp.dot(w2t_ref[...], h_t.astype(jnp.bfloat16),
                   preferred_element_type=jnp.float32)
    xj_t = jnp.transpose(xj_ref[...])[:m_in, :]
    acc = jnp.dot(b2t_ref[...], xj_t, preferred_element_type=jnp.float32)
    for i in range(m_in):
        acc = acc + xj_t[i:i + 1, :] * we_t[i * m_out:(i + 1) * m_out, :]
    o_ref[...] = jnp.transpose(acc)


def _tc_msg(ea_aug_t, xj, w1, b1, w2, b2, m_in, m_out, blk):
    grid = E // blk
    w1t = jnp.concatenate([w1.T, b1.reshape(-1, 1)], axis=1)
    w2t = w2.T.astype(jnp.bfloat16)
    b2t = b2.reshape(m_in, m_out).T
    return pl.pallas_call(
        functools.partial(_msg_body, m_in, m_out),
        grid=(grid,),
        in_specs=[
            pl.BlockSpec((6, blk), lambda i: (0, i)),
            pl.BlockSpec((blk, xj.shape[1]), lambda i: (i, 0)),
            pl.BlockSpec(w1t.shape, lambda i: (0, 0)),
            pl.BlockSpec(w2t.shape, lambda i: (0, 0)),
            pl.BlockSpec(b2t.shape, lambda i: (0, 0)),
        ],
        out_specs=pl.BlockSpec((blk, m_out), lambda i: (i, 0)),
        out_shape=jax.ShapeDtypeStruct((E, m_out), jnp.float32),
    )(ea_aug_t, xj, w1t, w2t, b2t)


def _h1_body(agg_ref, x5_ref, root_ref, bias_ref, o_ref):
    a = agg_ref[0] + agg_ref[1]
    v = a + jnp.dot(x5_ref[...], root_ref[...],
                    preferred_element_type=jnp.float32) + bias_ref[...]
    o_ref[...] = _elu(v)


def _tc_h1(agg, x5, root, bias):
    blk = 1024
    grid = NPAD // blk
    return pl.pallas_call(
        _h1_body,
        grid=(grid,),
        in_specs=[
            pl.BlockSpec((NC, blk, 32), lambda i: (0, i, 0)),
            pl.BlockSpec((blk, 5), lambda i: (i, 0)),
            pl.BlockSpec((5, 32), lambda i: (0, 0)),
            pl.BlockSpec((1, 32), lambda i: (0, 0)),
        ],
        out_specs=pl.BlockSpec((blk, 32), lambda i: (i, 0)),
        out_shape=jax.ShapeDtypeStruct((NPAD, 32), jnp.float32),
    )(agg, x5, root, bias.reshape(1, 32))


def _pool1_body(agg_ref, h1_ref, root_ref, bias_ref, xc_ref, seg_ref, o_ref):
    i = pl.program_id(0)
    a = agg_ref[0] + agg_ref[1]
    h2 = _elu(a + jnp.dot(h1_ref[...], root_ref[...],
                          preferred_element_type=jnp.float32) + bias_ref[...])
    bn = h2.shape[0]
    ones = jnp.ones((bn, 1), jnp.float32)
    feat = jnp.concatenate([h2, xc_ref[...], ones], axis=1)
    seg = seg_ref[0]
    iota = lax.broadcasted_iota(jnp.int32, (SUB, bn), 0)
    oh = (iota == seg).astype(jnp.float32)

    @pl.when(i == 0)
    def _():
        o_ref[...] = jnp.zeros_like(o_ref)

    o_ref[...] += jnp.dot(oh, feat, preferred_element_type=jnp.float32)


def _tc_pool1(agg2, h1, root2, bias2, xc, seg3d):
    blk = 1024
    grid = NPAD // blk
    return pl.pallas_call(
        _pool1_body,
        grid=(grid,),
        in_specs=[
            pl.BlockSpec((NC, blk, 64), lambda i: (0, i, 0)),
            pl.BlockSpec((blk, 32), lambda i: (i, 0)),
            pl.BlockSpec((32, 64), lambda i: (0, 0)),
            pl.BlockSpec((1, 64), lambda i: (0, 0)),
            pl.BlockSpec((blk, 11), lambda i: (i, 0)),
            pl.BlockSpec((1, 1, blk), lambda i: (i, 0, 0)),
        ],
        out_specs=pl.BlockSpec((SUB, 76), lambda i: (0, 0)),
        out_shape=jax.ShapeDtypeStruct((SUB, 76), jnp.float32),
    )(agg2, h1, root2, bias2.reshape(1, 64), xc, seg3d)


def _final_body(ss_ref, s2g_ref, w1_ref, b1_ref, w2_ref, b2_ref, w3_ref, b3_ref,
                o_ref):
    ss = ss_ref[...]
    cnt1 = ss[:, 75:76]
    mean1 = ss / jnp.maximum(cnt1, 1.0)
    s2g = s2g_ref[...]
    iota = lax.broadcasted_iota(jnp.int32, (G, SUB), 0)
    oh = (iota == s2g).astype(jnp.float32)
    gsum = jnp.dot(oh, mean1, preferred_element_type=jnp.float32)
    cnt2 = jnp.sum(oh, axis=1, keepdims=True)
    gmean = gsum / jnp.maximum(cnt2, 1.0)
    h = gmean[:, :75]
    h = _elu(jnp.dot(h, w1_ref[...], preferred_element_type=jnp.float32) + b1_ref[...])
    h = _elu(jnp.dot(h, w2_ref[...], preferred_element_type=jnp.float32) + b2_ref[...])
    o_ref[...] = jnp.dot(h, w3_ref[...], preferred_element_type=jnp.float32) + b3_ref[...]


def _tc_final(sub_sums, s2g, fc1_W, fc1_b, fc2_W, fc2_b, fc3_W, fc3_b):
    return pl.pallas_call(
        _final_body,
        out_shape=jax.ShapeDtypeStruct((G, 1), jnp.float32),
    )(sub_sums, s2g.reshape(1, SUB), fc1_W, fc1_b.reshape(1, 32),
      fc2_W, fc2_b.reshape(1, 16), fc3_W, fc3_b.reshape(1, 1))


def kernel(x, edge_index, edge_attr, node_to_subgraph, subgraph_to_graph,
           nn1_W1, nn1_b1, nn1_W2, nn1_b2, root1, bias1,
           nn2_W1, nn2_b1, nn2_W2, nn2_b2, root2, bias2,
           fc1_W, fc1_b, fc2_W, fc2_b, fc3_W, fc3_b):
    src = edge_index[0]
    dst = edge_index[1]
    x_pad = jnp.pad(x, ((0, NPAD - N), (0, 0)))
    x5 = x_pad[:, :CS]
    xc = x_pad[:, CS:]
    ea_aug_t = jnp.concatenate(
        [edge_attr.T, jnp.ones((1, E), jnp.float32)], axis=0)

    xj1 = _sc_gather(x_pad, src, FEAT)
    msg1 = _tc_msg(ea_aug_t, xj1, nn1_W1, nn1_b1, nn1_W2, nn1_b2, CS, 32, 1280)
    agg1 = _sc_scatter_add(msg1, dst, jnp.zeros((NPAD, 32), jnp.float32), 32)
    h1 = _tc_h1(agg1, x5, root1, bias1)

    xj2 = _sc_gather(h1, src, 32)
    msg2 = _tc_msg(ea_aug_t, xj2, nn2_W1, nn2_b1, nn2_W2, nn2_b2, 32, 64, 1280)
    agg2 = _sc_scatter_add(msg2, dst, jnp.zeros((NPAD, 64), jnp.float32), 64)

    seg = jnp.concatenate(
        [node_to_subgraph, jnp.full((NPAD - N,), SUB, jnp.int32)]).reshape(
            NPAD // 1024, 1, 1024)
    sub_sums = _tc_pool1(agg2, h1, root2, bias2, xc, seg)
    out = _tc_final(sub_sums, subgraph_to_graph,
                    fc1_W, fc1_b, fc2_W, fc2_b, fc3_W, fc3_b)
    return out.reshape(-1)


# speedup vs baseline: 2.1263x; 1.0350x over previous
import functools

import jax
import jax.numpy as jnp
from jax import lax
from jax.experimental import pallas as pl
from jax.experimental.pallas import tpu as pltpu
import jax.experimental.pallas.tpu_sc as plsc

N = 10000
E = 160000
SUB = 1000
G = 64
FEAT = 16
CS = 5

NPAD = 10240
NC = 2
NS = 16
NW = NC * NS
EPT = E // NW
CH = 1000
NCHUNK = EPT // CH
RPT = NPAD // NS


def _elu(v):
    return jnp.where(v > 0, v, jnp.exp(v) - 1.0)


def _sc_gather(table, idx, d):
    mesh = plsc.VectorSubcoreMesh(core_axis_name="c", subcore_axis_name="s")

    @functools.partial(
        pl.kernel,
        out_type=jax.ShapeDtypeStruct((E, d), jnp.float32),
        mesh=mesh,
        scratch_types=[
            pltpu.VMEM((CH,), jnp.int32),
            pltpu.VMEM((CH, d), jnp.float32),
            pltpu.SemaphoreType.DMA,
        ],
        compiler_params=pltpu.CompilerParams(use_tc_tiling_on_sc=False),
    )
    def k(table_hbm, idx_hbm, out_hbm, idx_v, rows_v, sem):
        wid = lax.axis_index("s") * NC + lax.axis_index("c")
        base = wid * EPT
        for j in range(NCHUNK):
            off = base + j * CH
            pltpu.sync_copy(idx_hbm.at[pl.ds(off, CH)], idx_v)
            pltpu.async_copy(table_hbm.at[idx_v], rows_v, sem).wait()
            pltpu.sync_copy(rows_v, out_hbm.at[pl.ds(off, CH)])

    return k(table, idx)


def _sc_scatter_add(msg, dst, zeros, d):
    mesh = plsc.VectorSubcoreMesh(core_axis_name="c", subcore_axis_name="s")

    @functools.partial(
        pl.kernel,
        out_type=jax.ShapeDtypeStruct((NC, NPAD, d), jnp.float32),
        mesh=mesh,
        scratch_types=[
            pltpu.VMEM((CH,), jnp.int32),
            pltpu.VMEM((CH, d), jnp.float32),
            pltpu.VMEM_SHARED((NPAD, d), jnp.float32),
            pltpu.SemaphoreType.DMA,
        ],
        compiler_params=pltpu.CompilerParams(use_tc_tiling_on_sc=False),
    )
    def k(msg_hbm, dst_hbm, z_hbm, out_hbm, idx_v, rows_v, acc, sem):
        c = lax.axis_index("c")
        s = lax.axis_index("s")
        rbase = s * RPT
        pltpu.sync_copy(z_hbm.at[pl.ds(rbase, RPT)], acc.at[pl.ds(rbase, RPT)])
        plsc.subcore_barrier()
        base = (s * NC + c) * EPT
        for j in range(NCHUNK):
            off = base + j * CH
            pltpu.sync_copy(dst_hbm.at[pl.ds(off, CH)], idx_v)
            pltpu.sync_copy(msg_hbm.at[pl.ds(off, CH)], rows_v)
            pltpu.sync_copy(rows_v, acc.at[idx_v], add=True)
        plsc.subcore_barrier()
        pltpu.sync_copy(acc.at[pl.ds(rbase, RPT)], out_hbm.at[c, pl.ds(rbase, RPT)])

    return k(msg, dst, zeros)


def _msg_body(m_in, m_out, eat_ref, xj_ref, w1t_ref, w2t_ref, b2t_ref, o_ref):
    h_t = jnp.maximum(jnp.dot(w1t_ref[...], eat_ref[...],
                              preferred_element_type=jnp.float32), 0.0)
    we_t = jnp.dot(w2t_ref[...], h_t.astype(jnp.bfloat16),
                   preferred_element_type=jnp.float32)
    xj_t = jnp.transpose(xj_ref[...])[:m_in, :]
    acc = jnp.dot(b2t_ref[...], xj_t, preferred_element_type=jnp.float32)
    for i in range(m_in):
        acc = acc + xj_t[i:i + 1, :] * we_t[i * m_out:(i + 1) * m_out, :]
    o_ref[...] = jnp.transpose(acc)


def _tc_msg(ea_aug_t, xj, w1, b1, w2, b2, m_in, m_out, blk):
    grid = E // blk
    w1t = jnp.concatenate([w1.T, b1.reshape(-1, 1)], axis=1)
    w2t = w2.T.astype(jnp.bfloat16)
    b2t = b2.reshape(m_in, m_out).T
    return pl.pallas_call(
        functools.partial(_msg_body, m_in, m_out),
        grid=(grid,),
        in_specs=[
            pl.BlockSpec((6, blk), lambda i: (0, i)),
            pl.BlockSpec((blk, xj.shape[1]), lambda i: (i, 0)),
            pl.BlockSpec(w1t.shape, lambda i: (0, 0)),
            pl.BlockSpec(w2t.shape, lambda i: (0, 0)),
            pl.BlockSpec(b2t.shape, lambda i: (0, 0)),
        ],
        out_specs=pl.BlockSpec((blk, m_out), lambda i: (i, 0)),
        out_shape=jax.ShapeDtypeStruct((E, m_out), jnp.float32),
    )(ea_aug_t, xj, w1t, w2t, b2t)


def _h1_body(agg_ref, x5_ref, root_ref, bias_ref, o_ref):
    a = agg_ref[0] + agg_ref[1]
    v = a + jnp.dot(x5_ref[...], root_ref[...],
                    preferred_element_type=jnp.float32) + bias_ref[...]
    o_ref[...] = _elu(v)


def _tc_h1(agg, x5, root, bias):
    blk = 1024
    grid = NPAD // blk
    return pl.pallas_call(
        _h1_body,
        grid=(grid,),
        in_specs=[
            pl.BlockSpec((NC, blk, 32), lambda i: (0, i, 0)),
            pl.BlockSpec((blk, 5), lambda i: (i, 0)),
            pl.BlockSpec((5, 32), lambda i: (0, 0)),
            pl.BlockSpec((1, 32), lambda i: (0, 0)),
        ],
        out_specs=pl.BlockSpec((blk, 32), lambda i: (i, 0)),
        out_shape=jax.ShapeDtypeStruct((NPAD, 32), jnp.float32),
    )(agg, x5, root, bias.reshape(1, 32))


def _pool1_body(agg_ref, h1_ref, root_ref, bias_ref, xc_ref, seg_ref, o_ref):
    i = pl.program_id(0)
    a = agg_ref[0] + agg_ref[1]
    h2 = _elu(a + jnp.dot(h1_ref[...], root_ref[...],
                          preferred_element_type=jnp.float32) + bias_ref[...])
    bn = h2.shape[0]
    ones = jnp.ones((bn, 1), jnp.float32)
    feat = jnp.concatenate([h2, xc_ref[...], ones], axis=1)
    seg = seg_ref[0]
    iota = lax.broadcasted_iota(jnp.int32, (SUB, bn), 0)
    oh = (iota == seg).astype(jnp.float32)

    @pl.when(i == 0)
    def _():
        o_ref[...] = jnp.zeros_like(o_ref)

    o_ref[...] += jnp.dot(oh, feat, preferred_element_type=jnp.float32)


def _tc_pool1(agg2, h1, root2, bias2, xc, seg3d):
    blk = 1024
    grid = NPAD // blk
    return pl.pallas_call(
        _pool1_body,
        grid=(grid,),
        in_specs=[
            pl.BlockSpec((NC, blk, 64), lambda i: (0, i, 0)),
            pl.BlockSpec((blk, 32), lambda i: (i, 0)),
            pl.BlockSpec((32, 64), lambda i: (0, 0)),
            pl.BlockSpec((1, 64), lambda i: (0, 0)),
            pl.BlockSpec((blk, 11), lambda i: (i, 0)),
            pl.BlockSpec((1, 1, blk), lambda i: (i, 0, 0)),
        ],
        out_specs=pl.BlockSpec((SUB, 76), lambda i: (0, 0)),
        out_shape=jax.ShapeDtypeStruct((SUB, 76), jnp.float32),
    )(agg2, h1, root2, bias2.reshape(1, 64), xc, seg3d)


def _final_body(ss_ref, s2g_ref, w1_ref, b1_ref, w2_ref, b2_ref, w3_ref, b3_ref,
                o_ref):
    ss = ss_ref[...]
    cnt1 = ss[:, 75:76]
    mean1 = ss / jnp.maximum(cnt1, 1.0)
    s2g = s2g_ref[...]
    iota = lax.broadcasted_iota(jnp.int32, (G, SUB), 0)
    oh = (iota == s2g).astype(jnp.float32)
    gsum = jnp.dot(oh, mean1, preferred_element_type=jnp.float32)
    cnt2 = jnp.sum(oh, axis=1, keepdims=True)
    gmean = gsum / jnp.maximum(cnt2, 1.0)
    h = gmean[:, :75]
    h = _elu(jnp.dot(h, w1_ref[...], preferred_element_type=jnp.float32) + b1_ref[...])
    h = _elu(jnp.dot(h, w2_ref[...], preferred_element_type=jnp.float32) + b2_ref[...])
    o_ref[...] = jnp.dot(h, w3_ref[...], preferred_element_type=jnp.float32) + b3_ref[...]


def _tc_final(sub_sums, s2g, fc1_W, fc1_b, fc2_W, fc2_b, fc3_W, fc3_b):
    return pl.pallas_call(
        _final_body,
        out_shape=jax.ShapeDtypeStruct((G, 1), jnp.float32),
    )(sub_sums, s2g.reshape(1, SUB), fc1_W, fc1_b.reshape(1, 32),
      fc2_W, fc2_b.reshape(1, 16), fc3_W, fc3_b.reshape(1, 1))


def kernel(x, edge_index, edge_attr, node_to_subgraph, subgraph_to_graph,
           nn1_W1, nn1_b1, nn1_W2, nn1_b2, root1, bias1,
           nn2_W1, nn2_b1, nn2_W2, nn2_b2, root2, bias2,
           fc1_W, fc1_b, fc2_W, fc2_b, fc3_W, fc3_b):
    src = edge_index[0]
    dst = edge_index[1]
    x_pad = jnp.pad(x, ((0, NPAD - N), (0, 0)))
    x5 = x_pad[:, :CS]
    xc = x_pad[:, CS:]
    ea_aug_t = jnp.concatenate(
        [edge_attr.T, jnp.ones((1, E), jnp.float32)], axis=0)

    xj1 = _sc_gather(x_pad, src, FEAT)
    msg1 = _tc_msg(ea_aug_t, xj1, nn1_W1, nn1_b1, nn1_W2, nn1_b2, CS, 32, 1280)
    agg1 = _sc_scatter_add(msg1, dst, jnp.zeros((NPAD, 32), jnp.float32), 32)
    h1 = _tc_h1(agg1, x5, root1, bias1)

    xj2 = _sc_gather(h1, src, 32)
    msg2 = _tc_msg(ea_aug_t, xj2, nn2_W1, nn2_b1, nn2_W2, nn2_b2, 32, 64, 3200)
    agg2 = _sc_scatter_add(msg2, dst, jnp.zeros((NPAD, 64), jnp.float32), 64)

    seg = jnp.concatenate(
        [node_to_subgraph, jnp.full((NPAD - N,), SUB, jnp.int32)]).reshape(
            NPAD // 1024, 1, 1024)
    sub_sums = _tc_pool1(agg2, h1, root2, bias2, xc, seg)
    out = _tc_final(sub_sums, subgraph_to_graph,
                    fc1_W, fc1_b, fc2_W, fc2_b, fc3_W, fc3_b)
    return out.reshape(-1)


# speedup vs baseline: 2.2469x; 1.0567x over previous
import functools

import jax
import jax.numpy as jnp
from jax import lax
from jax.experimental import pallas as pl
from jax.experimental.pallas import tpu as pltpu
import jax.experimental.pallas.tpu_sc as plsc

N = 10000
E = 160000
SUB = 1000
G = 64
FEAT = 16
CS = 5

NPAD = 10240
NC = 2
NS = 16
NW = NC * NS
EPT = E // NW
CH = 1000
NCHUNK = EPT // CH
RPT = NPAD // NS


def _elu(v):
    return jnp.where(v > 0, v, jnp.exp(v) - 1.0)


def _sc_gather(table, idx, d):
    mesh = plsc.VectorSubcoreMesh(core_axis_name="c", subcore_axis_name="s")

    @functools.partial(
        pl.kernel,
        out_type=jax.ShapeDtypeStruct((E, d), jnp.float32),
        mesh=mesh,
        scratch_types=[
            pltpu.VMEM((CH,), jnp.int32),
            pltpu.VMEM((CH, d), jnp.float32),
            pltpu.SemaphoreType.DMA,
        ],
        compiler_params=pltpu.CompilerParams(use_tc_tiling_on_sc=False),
    )
    def k(table_hbm, idx_hbm, out_hbm, idx_v, rows_v, sem):
        wid = lax.axis_index("s") * NC + lax.axis_index("c")
        base = wid * EPT
        for j in range(NCHUNK):
            off = base + j * CH
            pltpu.sync_copy(idx_hbm.at[pl.ds(off, CH)], idx_v)
            pltpu.async_copy(table_hbm.at[idx_v], rows_v, sem).wait()
            pltpu.sync_copy(rows_v, out_hbm.at[pl.ds(off, CH)])

    return k(table, idx)


def _sc_scatter_add(msg, dst, zeros, d):
    mesh = plsc.VectorSubcoreMesh(core_axis_name="c", subcore_axis_name="s")

    @functools.partial(
        pl.kernel,
        out_type=jax.ShapeDtypeStruct((NC, NPAD, d), jnp.float32),
        mesh=mesh,
        scratch_types=[
            pltpu.VMEM((CH,), jnp.int32),
            pltpu.VMEM((CH, d), jnp.float32),
            pltpu.VMEM_SHARED((NPAD, d), jnp.float32),
            pltpu.SemaphoreType.DMA,
        ],
        compiler_params=pltpu.CompilerParams(use_tc_tiling_on_sc=False),
    )
    def k(msg_hbm, dst_hbm, z_hbm, out_hbm, idx_v, rows_v, acc, sem):
        c = lax.axis_index("c")
        s = lax.axis_index("s")
        rbase = s * RPT
        pltpu.sync_copy(z_hbm.at[pl.ds(rbase, RPT)], acc.at[pl.ds(rbase, RPT)])
        plsc.subcore_barrier()
        base = (s * NC + c) * EPT
        for j in range(NCHUNK):
            off = base + j * CH
            pltpu.sync_copy(dst_hbm.at[pl.ds(off, CH)], idx_v)
            pltpu.sync_copy(msg_hbm.at[pl.ds(off, CH)], rows_v)
            pltpu.sync_copy(rows_v, acc.at[idx_v], add=True)
        plsc.subcore_barrier()
        pltpu.sync_copy(acc.at[pl.ds(rbase, RPT)], out_hbm.at[c, pl.ds(rbase, RPT)])

    return k(msg, dst, zeros)


def _msg_body(m_in, m_out, eat_ref, xj_ref, w1t_ref, w2t_ref, b2t_ref, o_ref):
    h_t = jnp.maximum(jnp.dot(w1t_ref[...], eat_ref[...],
                              preferred_element_type=jnp.float32), 0.0)
    we_t = jnp.dot(w2t_ref[...], h_t.astype(jnp.bfloat16),
                   preferred_element_type=jnp.float32)
    xj_t = jnp.transpose(xj_ref[...])[:m_in, :]
    acc = jnp.dot(b2t_ref[...], xj_t, preferred_element_type=jnp.float32)
    for i in range(m_in):
        acc = acc + xj_t[i:i + 1, :] * we_t[i * m_out:(i + 1) * m_out, :]
    o_ref[...] = jnp.transpose(acc)


def _tc_msg(ea_aug_t, xj, w1, b1, w2, b2, m_in, m_out, blk):
    grid = E // blk
    w1t = jnp.concatenate([w1.T, b1.reshape(-1, 1)], axis=1)
    w2t = w2.T.astype(jnp.bfloat16)
    b2t = b2.reshape(m_in, m_out).T
    return pl.pallas_call(
        functools.partial(_msg_body, m_in, m_out),
        grid=(grid,),
        in_specs=[
            pl.BlockSpec((6, blk), lambda i: (0, i)),
            pl.BlockSpec((blk, xj.shape[1]), lambda i: (i, 0)),
            pl.BlockSpec(w1t.shape, lambda i: (0, 0)),
            pl.BlockSpec(w2t.shape, lambda i: (0, 0)),
            pl.BlockSpec(b2t.shape, lambda i: (0, 0)),
        ],
        out_specs=pl.BlockSpec((blk, m_out), lambda i: (i, 0)),
        out_shape=jax.ShapeDtypeStruct((E, m_out), jnp.float32),
    )(ea_aug_t, xj, w1t, w2t, b2t)


def _h1_body(agg_ref, x5_ref, root_ref, bias_ref, o_ref):
    a = agg_ref[0] + agg_ref[1]
    v = a + jnp.dot(x5_ref[...], root_ref[...],
                    preferred_element_type=jnp.float32) + bias_ref[...]
    o_ref[...] = _elu(v)


def _tc_h1(agg, x5, root, bias):
    blk = 1024
    grid = NPAD // blk
    return pl.pallas_call(
        _h1_body,
        grid=(grid,),
        in_specs=[
            pl.BlockSpec((NC, blk, 32), lambda i: (0, i, 0)),
            pl.BlockSpec((blk, 5), lambda i: (i, 0)),
            pl.BlockSpec((5, 32), lambda i: (0, 0)),
            pl.BlockSpec((1, 32), lambda i: (0, 0)),
        ],
        out_specs=pl.BlockSpec((blk, 32), lambda i: (i, 0)),
        out_shape=jax.ShapeDtypeStruct((NPAD, 32), jnp.float32),
    )(agg, x5, root, bias.reshape(1, 32))


def _pool1_body(agg_ref, h1_ref, root_ref, bias_ref, xc_ref, seg_ref, o_ref):
    i = pl.program_id(0)
    a = agg_ref[0] + agg_ref[1]
    h2 = _elu(a + jnp.dot(h1_ref[...], root_ref[...],
                          preferred_element_type=jnp.float32) + bias_ref[...])
    bn = h2.shape[0]
    ones = jnp.ones((bn, 1), jnp.float32)
    feat = jnp.concatenate([h2, xc_ref[...], ones], axis=1)
    seg = seg_ref[0]
    iota = lax.broadcasted_iota(jnp.int32, (SUB, bn), 0)
    oh = (iota == seg).astype(jnp.float32)

    @pl.when(i == 0)
    def _():
        o_ref[...] = jnp.zeros_like(o_ref)

    o_ref[...] += jnp.dot(oh, feat, preferred_element_type=jnp.float32)


def _tc_pool1(agg2, h1, root2, bias2, xc, seg3d):
    blk = 1024
    grid = NPAD // blk
    return pl.pallas_call(
        _pool1_body,
        grid=(grid,),
        in_specs=[
            pl.BlockSpec((NC, blk, 64), lambda i: (0, i, 0)),
            pl.BlockSpec((blk, 32), lambda i: (i, 0)),
            pl.BlockSpec((32, 64), lambda i: (0, 0)),
            pl.BlockSpec((1, 64), lambda i: (0, 0)),
            pl.BlockSpec((blk, 11), lambda i: (i, 0)),
            pl.BlockSpec((1, 1, blk), lambda i: (i, 0, 0)),
        ],
        out_specs=pl.BlockSpec((SUB, 76), lambda i: (0, 0)),
        out_shape=jax.ShapeDtypeStruct((SUB, 76), jnp.float32),
    )(agg2, h1, root2, bias2.reshape(1, 64), xc, seg3d)


def _final_body(ss_ref, s2g_ref, w1_ref, b1_ref, w2_ref, b2_ref, w3_ref, b3_ref,
                o_ref):
    ss = ss_ref[...]
    cnt1 = ss[:, 75:76]
    mean1 = ss / jnp.maximum(cnt1, 1.0)
    s2g = s2g_ref[...]
    iota = lax.broadcasted_iota(jnp.int32, (G, SUB), 0)
    oh = (iota == s2g).astype(jnp.float32)
    gsum = jnp.dot(oh, mean1, preferred_element_type=jnp.float32)
    cnt2 = jnp.sum(oh, axis=1, keepdims=True)
    gmean = gsum / jnp.maximum(cnt2, 1.0)
    h = gmean[:, :75]
    h = _elu(jnp.dot(h, w1_ref[...], preferred_element_type=jnp.float32) + b1_ref[...])
    h = _elu(jnp.dot(h, w2_ref[...], preferred_element_type=jnp.float32) + b2_ref[...])
    o_ref[...] = jnp.dot(h, w3_ref[...], preferred_element_type=jnp.float32) + b3_ref[...]


def _tc_final(sub_sums, s2g, fc1_W, fc1_b, fc2_W, fc2_b, fc3_W, fc3_b):
    return pl.pallas_call(
        _final_body,
        out_shape=jax.ShapeDtypeStruct((G, 1), jnp.float32),
    )(sub_sums, s2g.reshape(1, SUB), fc1_W, fc1_b.reshape(1, 32),
      fc2_W, fc2_b.reshape(1, 16), fc3_W, fc3_b.reshape(1, 1))


def kernel(x, edge_index, edge_attr, node_to_subgraph, subgraph_to_graph,
           nn1_W1, nn1_b1, nn1_W2, nn1_b2, root1, bias1,
           nn2_W1, nn2_b1, nn2_W2, nn2_b2, root2, bias2,
           fc1_W, fc1_b, fc2_W, fc2_b, fc3_W, fc3_b):
    src = edge_index[0]
    dst = edge_index[1]
    x_pad = jnp.pad(x, ((0, NPAD - N), (0, 0)))
    x5 = x_pad[:, :CS]
    xc = x_pad[:, CS:]
    ea_aug_t = jnp.concatenate(
        [edge_attr.T, jnp.ones((1, E), jnp.float32)], axis=0)

    xj1 = _sc_gather(x_pad, src, FEAT)
    msg1 = _tc_msg(ea_aug_t, xj1, nn1_W1, nn1_b1, nn1_W2, nn1_b2, CS, 32, 3200)
    agg1 = _sc_scatter_add(msg1, dst, jnp.zeros((NPAD, 32), jnp.float32), 32)
    h1 = _tc_h1(agg1, x5, root1, bias1)

    xj2 = _sc_gather(h1, src, 32)
    msg2 = _tc_msg(ea_aug_t, xj2, nn2_W1, nn2_b1, nn2_W2, nn2_b2, 32, 64, 3200)
    agg2 = _sc_scatter_add(msg2, dst, jnp.zeros((NPAD, 64), jnp.float32), 64)

    seg = jnp.concatenate(
        [node_to_subgraph, jnp.full((NPAD - N,), SUB, jnp.int32)]).reshape(
            NPAD // 1024, 1, 1024)
    sub_sums = _tc_pool1(agg2, h1, root2, bias2, xc, seg)
    out = _tc_final(sub_sums, subgraph_to_graph,
                    fc1_W, fc1_b, fc2_W, fc2_b, fc3_W, fc3_b)
    return out.reshape(-1)


# speedup vs baseline: 2.2863x; 1.0175x over previous
import functools

import jax
import jax.numpy as jnp
from jax import lax
from jax.experimental import pallas as pl
from jax.experimental.pallas import tpu as pltpu
import jax.experimental.pallas.tpu_sc as plsc

N = 10000
E = 160000
SUB = 1000
G = 64
FEAT = 16
CS = 5

NPAD = 10240
NC = 2
NS = 16
NW = NC * NS
EPT = E // NW
CH = 1000
NCHUNK = EPT // CH
RPT = NPAD // NS


def _elu(v):
    return jnp.where(v > 0, v, jnp.exp(v) - 1.0)


def _sc_gather(table, idx, d):
    mesh = plsc.VectorSubcoreMesh(core_axis_name="c", subcore_axis_name="s")

    @functools.partial(
        pl.kernel,
        out_type=jax.ShapeDtypeStruct((E, d), jnp.float32),
        mesh=mesh,
        scratch_types=[
            pltpu.VMEM((CH,), jnp.int32),
            pltpu.VMEM((CH, d), jnp.float32),
            pltpu.SemaphoreType.DMA,
        ],
        compiler_params=pltpu.CompilerParams(use_tc_tiling_on_sc=False),
    )
    def k(table_hbm, idx_hbm, out_hbm, idx_v, rows_v, sem):
        wid = lax.axis_index("s") * NC + lax.axis_index("c")
        base = wid * EPT
        for j in range(NCHUNK):
            off = base + j * CH
            pltpu.sync_copy(idx_hbm.at[pl.ds(off, CH)], idx_v)
            pltpu.async_copy(table_hbm.at[idx_v], rows_v, sem).wait()
            pltpu.sync_copy(rows_v, out_hbm.at[pl.ds(off, CH)])

    return k(table, idx)


def _sc_scatter_add(msg, dst, zeros, d):
    mesh = plsc.VectorSubcoreMesh(core_axis_name="c", subcore_axis_name="s")

    @functools.partial(
        pl.kernel,
        out_type=jax.ShapeDtypeStruct((NC, NPAD, d), jnp.float32),
        mesh=mesh,
        scratch_types=[
            pltpu.VMEM((CH,), jnp.int32),
            pltpu.VMEM((CH, d), jnp.float32),
            pltpu.VMEM_SHARED((NPAD, d), jnp.float32),
            pltpu.SemaphoreType.DMA,
        ],
        compiler_params=pltpu.CompilerParams(use_tc_tiling_on_sc=False),
    )
    def k(msg_hbm, dst_hbm, z_hbm, out_hbm, idx_v, rows_v, acc, sem):
        c = lax.axis_index("c")
        s = lax.axis_index("s")
        rbase = s * RPT
        pltpu.sync_copy(z_hbm.at[pl.ds(rbase, RPT)], acc.at[pl.ds(rbase, RPT)])
        plsc.subcore_barrier()
        base = (s * NC + c) * EPT
        for j in range(NCHUNK):
            off = base + j * CH
            pltpu.sync_copy(dst_hbm.at[pl.ds(off, CH)], idx_v)
            pltpu.sync_copy(msg_hbm.at[pl.ds(off, CH)], rows_v)
            pltpu.sync_copy(rows_v, acc.at[idx_v], add=True)
        plsc.subcore_barrier()
        pltpu.sync_copy(acc.at[pl.ds(rbase, RPT)], out_hbm.at[c, pl.ds(rbase, RPT)])

    return k(msg, dst, zeros)


def _msg_body(m_in, m_out, eat_ref, xj_ref, w1t_ref, w2t_ref, b2t_ref, o_ref):
    h_t = jnp.maximum(jnp.dot(w1t_ref[...], eat_ref[...],
                              preferred_element_type=jnp.float32), 0.0)
    we_t = jnp.dot(w2t_ref[...], h_t.astype(jnp.bfloat16),
                   preferred_element_type=jnp.float32)
    xj_t = jnp.transpose(xj_ref[...])[:m_in, :]
    acc = jnp.dot(b2t_ref[...], xj_t, preferred_element_type=jnp.float32)
    for i in range(m_in):
        acc = acc + xj_t[i:i + 1, :] * we_t[i * m_out:(i + 1) * m_out, :]
    o_ref[...] = jnp.transpose(acc)


def _tc_msg(ea_aug_t, xj, w1, b1, w2, b2, m_in, m_out, blk):
    grid = E // blk
    w1t = jnp.concatenate([w1.T, b1.reshape(-1, 1)], axis=1)
    w2t = w2.T.astype(jnp.bfloat16)
    b2t = b2.reshape(m_in, m_out).T
    return pl.pallas_call(
        functools.partial(_msg_body, m_in, m_out),
        grid=(grid,),
        in_specs=[
            pl.BlockSpec((6, blk), lambda i: (0, i)),
            pl.BlockSpec((blk, xj.shape[1]), lambda i: (i, 0)),
            pl.BlockSpec(w1t.shape, lambda i: (0, 0)),
            pl.BlockSpec(w2t.shape, lambda i: (0, 0)),
            pl.BlockSpec(b2t.shape, lambda i: (0, 0)),
        ],
        out_specs=pl.BlockSpec((blk, m_out), lambda i: (i, 0)),
        out_shape=jax.ShapeDtypeStruct((E, m_out), jnp.float32),
    )(ea_aug_t, xj, w1t, w2t, b2t)


def _h1_body(agg_ref, x5_ref, root_ref, bias_ref, o_ref):
    a = agg_ref[0] + agg_ref[1]
    v = a + jnp.dot(x5_ref[...], root_ref[...],
                    preferred_element_type=jnp.float32) + bias_ref[...]
    o_ref[...] = _elu(v)


def _tc_h1(agg, x5, root, bias):
    blk = 1024
    grid = NPAD // blk
    return pl.pallas_call(
        _h1_body,
        grid=(grid,),
        in_specs=[
            pl.BlockSpec((NC, blk, 32), lambda i: (0, i, 0)),
            pl.BlockSpec((blk, 5), lambda i: (i, 0)),
            pl.BlockSpec((5, 32), lambda i: (0, 0)),
            pl.BlockSpec((1, 32), lambda i: (0, 0)),
        ],
        out_specs=pl.BlockSpec((blk, 32), lambda i: (i, 0)),
        out_shape=jax.ShapeDtypeStruct((NPAD, 32), jnp.float32),
    )(agg, x5, root, bias.reshape(1, 32))


def _pool1_body(agg_ref, h1_ref, root_ref, bias_ref, xc_ref, seg_ref, o_ref):
    i = pl.program_id(0)
    a = agg_ref[0] + agg_ref[1]
    h2 = _elu(a + jnp.dot(h1_ref[...], root_ref[...],
                          preferred_element_type=jnp.float32) + bias_ref[...])
    bn = h2.shape[0]
    ones = jnp.ones((bn, 1), jnp.float32)
    feat = jnp.concatenate([h2, xc_ref[...], ones], axis=1)
    seg = seg_ref[0]
    iota = lax.broadcasted_iota(jnp.int32, (SUB, bn), 0)
    oh = (iota == seg).astype(jnp.float32)

    @pl.when(i == 0)
    def _():
        o_ref[...] = jnp.zeros_like(o_ref)

    o_ref[...] += jnp.dot(oh, feat, preferred_element_type=jnp.float32)


def _tc_pool1(agg2, h1, root2, bias2, xc, seg3d):
    blk = 1024
    grid = NPAD // blk
    return pl.pallas_call(
        _pool1_body,
        grid=(grid,),
        in_specs=[
            pl.BlockSpec((NC, blk, 64), lambda i: (0, i, 0)),
            pl.BlockSpec((blk, 32), lambda i: (i, 0)),
            pl.BlockSpec((32, 64), lambda i: (0, 0)),
            pl.BlockSpec((1, 64), lambda i: (0, 0)),
            pl.BlockSpec((blk, 11), lambda i: (i, 0)),
            pl.BlockSpec((1, 1, blk), lambda i: (i, 0, 0)),
        ],
        out_specs=pl.BlockSpec((SUB, 76), lambda i: (0, 0)),
        out_shape=jax.ShapeDtypeStruct((SUB, 76), jnp.float32),
    )(agg2, h1, root2, bias2.reshape(1, 64), xc, seg3d)


def _final_body(ss_ref, s2g_ref, w1_ref, b1_ref, w2_ref, b2_ref, w3_ref, b3_ref,
                o_ref):
    ss = ss_ref[...]
    cnt1 = ss[:, 75:76]
    mean1 = ss / jnp.maximum(cnt1, 1.0)
    s2g = s2g_ref[...]
    iota = lax.broadcasted_iota(jnp.int32, (G, SUB), 0)
    oh = (iota == s2g).astype(jnp.float32)
    gsum = jnp.dot(oh, mean1, preferred_element_type=jnp.float32)
    cnt2 = jnp.sum(oh, axis=1, keepdims=True)
    gmean = gsum / jnp.maximum(cnt2, 1.0)
    h = gmean[:, :75]
    h = _elu(jnp.dot(h, w1_ref[...], preferred_element_type=jnp.float32) + b1_ref[...])
    h = _elu(jnp.dot(h, w2_ref[...], preferred_element_type=jnp.float32) + b2_ref[...])
    o_ref[...] = jnp.dot(h, w3_ref[...], preferred_element_type=jnp.float32) + b3_ref[...]


def _tc_final(sub_sums, s2g, fc1_W, fc1_b, fc2_W, fc2_b, fc3_W, fc3_b):
    return pl.pallas_call(
        _final_body,
        out_shape=jax.ShapeDtypeStruct((G, 1), jnp.float32),
    )(sub_sums, s2g.reshape(1, SUB), fc1_W, fc1_b.reshape(1, 32),
      fc2_W, fc2_b.reshape(1, 16), fc3_W, fc3_b.reshape(1, 1))


def kernel(x, edge_index, edge_attr, node_to_subgraph, subgraph_to_graph,
           nn1_W1, nn1_b1, nn1_W2, nn1_b2, root1, bias1,
           nn2_W1, nn2_b1, nn2_W2, nn2_b2, root2, bias2,
           fc1_W, fc1_b, fc2_W, fc2_b, fc3_W, fc3_b):
    src = edge_index[0]
    dst = edge_index[1]
    x_pad = jnp.pad(x, ((0, NPAD - N), (0, 0)))
    x5 = x_pad[:, :CS]
    xc = x_pad[:, CS:]
    ea_aug_t = jnp.concatenate(
        [edge_attr.T, jnp.ones((1, E), jnp.float32)], axis=0)

    xj1 = _sc_gather(x_pad, src, FEAT)
    msg1 = _tc_msg(ea_aug_t, xj1, nn1_W1, nn1_b1, nn1_W2, nn1_b2, CS, 32, 6400)
    agg1 = _sc_scatter_add(msg1, dst, jnp.zeros((NPAD, 32), jnp.float32), 32)
    h1 = _tc_h1(agg1, x5, root1, bias1)

    xj2 = _sc_gather(h1, src, 32)
    msg2 = _tc_msg(ea_aug_t, xj2, nn2_W1, nn2_b1, nn2_W2, nn2_b2, 32, 64, 3200)
    agg2 = _sc_scatter_add(msg2, dst, jnp.zeros((NPAD, 64), jnp.float32), 64)

    seg = jnp.concatenate(
        [node_to_subgraph, jnp.full((NPAD - N,), SUB, jnp.int32)]).reshape(
            NPAD // 1024, 1, 1024)
    sub_sums = _tc_pool1(agg2, h1, root2, bias2, xc, seg)
    out = _tc_final(sub_sums, subgraph_to_graph,
                    fc1_W, fc1_b, fc2_W, fc2_b, fc3_W, fc3_b)
    return out.reshape(-1)


# speedup vs baseline: 2.3426x; 1.0247x over previous
import functools

import jax
import jax.numpy as jnp
from jax import lax
from jax.experimental import pallas as pl
from jax.experimental.pallas import tpu as pltpu
import jax.experimental.pallas.tpu_sc as plsc

N = 10000
E = 160000
SUB = 1000
G = 64
FEAT = 16
CS = 5

NPAD = 10240
NC = 2
NS = 16
NW = NC * NS
EPT = E // NW
CH = 1000
NCHUNK = EPT // CH
RPT = NPAD // NS


def _elu(v):
    return jnp.where(v > 0, v, jnp.exp(v) - 1.0)


def _sc_gather(table, idx, d):
    mesh = plsc.VectorSubcoreMesh(core_axis_name="c", subcore_axis_name="s")

    @functools.partial(
        pl.kernel,
        out_type=jax.ShapeDtypeStruct((E, d), jnp.float32),
        mesh=mesh,
        scratch_types=[
            pltpu.VMEM((EPT,), jnp.int32),
            pltpu.VMEM((2, CH, d), jnp.float32),
            pltpu.SemaphoreType.DMA,
            pltpu.SemaphoreType.DMA,
            pltpu.SemaphoreType.DMA,
            pltpu.SemaphoreType.DMA,
        ],
        compiler_params=pltpu.CompilerParams(use_tc_tiling_on_sc=False),
    )
    def k(table_hbm, idx_hbm, out_hbm, idx_v, rows_v, g0, g1, s0, s1):
        wid = lax.axis_index("s") * NC + lax.axis_index("c")
        base = wid * EPT
        gsem = (g0, g1)
        ssem = (s0, s1)
        pltpu.sync_copy(idx_hbm.at[pl.ds(base, EPT)], idx_v)
        gathers = [pltpu.async_copy(
            table_hbm.at[idx_v.at[pl.ds(0, CH)]], rows_v.at[0], gsem[0])]
        stores = []
        for j in range(NCHUNK):
            b = j % 2
            gathers[j].wait()
            if j >= 1:
                stores[j - 1].wait()
            if j + 1 < NCHUNK:
                gathers.append(pltpu.async_copy(
                    table_hbm.at[idx_v.at[pl.ds((j + 1) * CH, CH)]],
                    rows_v.at[1 - b], gsem[1 - b]))
            stores.append(pltpu.async_copy(
                rows_v.at[b], out_hbm.at[pl.ds(base + j * CH, CH)], ssem[b]))
        stores[NCHUNK - 1].wait()

    return k(table, idx)


def _sc_scatter_add(msg, dst, zeros, d):
    nbuf = 2 if d <= 32 else 1
    mesh = plsc.VectorSubcoreMesh(core_axis_name="c", subcore_axis_name="s")

    @functools.partial(
        pl.kernel,
        out_type=jax.ShapeDtypeStruct((NC, NPAD, d), jnp.float32),
        mesh=mesh,
        scratch_types=[
            pltpu.VMEM((nbuf, CH), jnp.int32),
            pltpu.VMEM((nbuf, CH, d), jnp.float32),
            pltpu.VMEM_SHARED((NPAD, d), jnp.float32),
            pltpu.SemaphoreType.DMA,
            pltpu.SemaphoreType.DMA,
            pltpu.SemaphoreType.DMA,
        ],
        compiler_params=pltpu.CompilerParams(use_tc_tiling_on_sc=False),
    )
    def k(msg_hbm, dst_hbm, z_hbm, out_hbm, idx_v, rows_v, acc, isem, l0, l1):
        c = lax.axis_index("c")
        s = lax.axis_index("s")
        rbase = s * RPT
        lsem = (l0, l1)
        zcp = pltpu.async_copy(z_hbm.at[pl.ds(rbase, RPT)],
                               acc.at[pl.ds(rbase, RPT)], isem)
        base = (s * NC + c) * EPT
        iloads = [pltpu.async_copy(
            dst_hbm.at[pl.ds(base, CH)], idx_v.at[0], lsem[0])]
        loads = [pltpu.async_copy(
            msg_hbm.at[pl.ds(base, CH)], rows_v.at[0], lsem[0])]
        zcp.wait()
        plsc.subcore_barrier()
        for j in range(NCHUNK):
            b = (j % 2) if nbuf == 2 else 0
            nb = 1 - b if nbuf == 2 else 0
            iloads[j].wait()
            loads[j].wait()
            if nbuf == 1:
                pltpu.sync_copy(rows_v.at[b], acc.at[idx_v.at[b]], add=True)
            if j + 1 < NCHUNK:
                iloads.append(pltpu.async_copy(
                    dst_hbm.at[pl.ds(base + (j + 1) * CH, CH)],
                    idx_v.at[nb], lsem[nb]))
                loads.append(pltpu.async_copy(
                    msg_hbm.at[pl.ds(base + (j + 1) * CH, CH)],
                    rows_v.at[nb], lsem[nb]))
            if nbuf == 2:
                pltpu.sync_copy(rows_v.at[b], acc.at[idx_v.at[b]], add=True)
        plsc.subcore_barrier()
        pltpu.sync_copy(acc.at[pl.ds(rbase, RPT)], out_hbm.at[c, pl.ds(rbase, RPT)])

    return k(msg, dst, zeros)


def _msg_body(m_in, m_out, eat_ref, xj_ref, w1t_ref, w2t_ref, b2t_ref, o_ref):
    h_t = jnp.maximum(jnp.dot(w1t_ref[...], eat_ref[...],
                              preferred_element_type=jnp.float32), 0.0)
    we_t = jnp.dot(w2t_ref[...], h_t.astype(jnp.bfloat16),
                   preferred_element_type=jnp.float32)
    xj_t = jnp.transpose(xj_ref[...])[:m_in, :]
    acc = jnp.dot(b2t_ref[...], xj_t, preferred_element_type=jnp.float32)
    for i in range(m_in):
        acc = acc + xj_t[i:i + 1, :] * we_t[i * m_out:(i + 1) * m_out, :]
    o_ref[...] = jnp.transpose(acc)


def _tc_msg(ea_aug_t, xj, w1, b1, w2, b2, m_in, m_out, blk):
    grid = E // blk
    w1t = jnp.concatenate([w1.T, b1.reshape(-1, 1)], axis=1)
    w2t = w2.T.astype(jnp.bfloat16)
    b2t = b2.reshape(m_in, m_out).T
    return pl.pallas_call(
        functools.partial(_msg_body, m_in, m_out),
        grid=(grid,),
        in_specs=[
            pl.BlockSpec((6, blk), lambda i: (0, i)),
            pl.BlockSpec((blk, xj.shape[1]), lambda i: (i, 0)),
            pl.BlockSpec(w1t.shape, lambda i: (0, 0)),
            pl.BlockSpec(w2t.shape, lambda i: (0, 0)),
            pl.BlockSpec(b2t.shape, lambda i: (0, 0)),
        ],
        out_specs=pl.BlockSpec((blk, m_out), lambda i: (i, 0)),
        out_shape=jax.ShapeDtypeStruct((E, m_out), jnp.float32),
    )(ea_aug_t, xj, w1t, w2t, b2t)


def _h1_body(agg_ref, x5_ref, root_ref, bias_ref, o_ref):
    a = agg_ref[0] + agg_ref[1]
    v = a + jnp.dot(x5_ref[...], root_ref[...],
                    preferred_element_type=jnp.float32) + bias_ref[...]
    o_ref[...] = _elu(v)


def _tc_h1(agg, x5, root, bias):
    blk = 1024
    grid = NPAD // blk
    return pl.pallas_call(
        _h1_body,
        grid=(grid,),
        in_specs=[
            pl.BlockSpec((NC, blk, 32), lambda i: (0, i, 0)),
            pl.BlockSpec((blk, 5), lambda i: (i, 0)),
            pl.BlockSpec((5, 32), lambda i: (0, 0)),
            pl.BlockSpec((1, 32), lambda i: (0, 0)),
        ],
        out_specs=pl.BlockSpec((blk, 32), lambda i: (i, 0)),
        out_shape=jax.ShapeDtypeStruct((NPAD, 32), jnp.float32),
    )(agg, x5, root, bias.reshape(1, 32))


def _pool1_body(agg_ref, h1_ref, root_ref, bias_ref, xc_ref, seg_ref, o_ref):
    i = pl.program_id(0)
    a = agg_ref[0] + agg_ref[1]
    h2 = _elu(a + jnp.dot(h1_ref[...], root_ref[...],
                          preferred_element_type=jnp.float32) + bias_ref[...])
    bn = h2.shape[0]
    ones = jnp.ones((bn, 1), jnp.float32)
    feat = jnp.concatenate([h2, xc_ref[...], ones], axis=1)
    seg = seg_ref[0]
    iota = lax.broadcasted_iota(jnp.int32, (SUB, bn), 0)
    oh = (iota == seg).astype(jnp.float32)

    @pl.when(i == 0)
    def _():
        o_ref[...] = jnp.zeros_like(o_ref)

    o_ref[...] += jnp.dot(oh, feat, preferred_element_type=jnp.float32)


def _tc_pool1(agg2, h1, root2, bias2, xc, seg3d):
    blk = 1024
    grid = NPAD // blk
    return pl.pallas_call(
        _pool1_body,
        grid=(grid,),
        in_specs=[
            pl.BlockSpec((NC, blk, 64), lambda i: (0, i, 0)),
            pl.BlockSpec((blk, 32), lambda i: (i, 0)),
            pl.BlockSpec((32, 64), lambda i: (0, 0)),
            pl.BlockSpec((1, 64), lambda i: (0, 0)),
            pl.BlockSpec((blk, 11), lambda i: (i, 0)),
            pl.BlockSpec((1, 1, blk), lambda i: (i, 0, 0)),
        ],
        out_specs=pl.BlockSpec((SUB, 76), lambda i: (0, 0)),
        out_shape=jax.ShapeDtypeStruct((SUB, 76), jnp.float32),
    )(agg2, h1, root2, bias2.reshape(1, 64), xc, seg3d)


def _final_body(ss_ref, s2g_ref, w1_ref, b1_ref, w2_ref, b2_ref, w3_ref, b3_ref,
                o_ref):
    ss = ss_ref[...]
    cnt1 = ss[:, 75:76]
    mean1 = ss / jnp.maximum(cnt1, 1.0)
    s2g = s2g_ref[...]
    iota = lax.broadcasted_iota(jnp.int32, (G, SUB), 0)
    oh = (iota == s2g).astype(jnp.float32)
    gsum = jnp.dot(oh, mean1, preferred_element_type=jnp.float32)
    cnt2 = jnp.sum(oh, axis=1, keepdims=True)
    gmean = gsum / jnp.maximum(cnt2, 1.0)
    h = gmean[:, :75]
    h = _elu(jnp.dot(h, w1_ref[...], preferred_element_type=jnp.float32) + b1_ref[...])
    h = _elu(jnp.dot(h, w2_ref[...], preferred_element_type=jnp.float32) + b2_ref[...])
    o_ref[...] = jnp.dot(h, w3_ref[...], preferred_element_type=jnp.float32) + b3_ref[...]


def _tc_final(sub_sums, s2g, fc1_W, fc1_b, fc2_W, fc2_b, fc3_W, fc3_b):
    return pl.pallas_call(
        _final_body,
        out_shape=jax.ShapeDtypeStruct((G, 1), jnp.float32),
    )(sub_sums, s2g.reshape(1, SUB), fc1_W, fc1_b.reshape(1, 32),
      fc2_W, fc2_b.reshape(1, 16), fc3_W, fc3_b.reshape(1, 1))


def kernel(x, edge_index, edge_attr, node_to_subgraph, subgraph_to_graph,
           nn1_W1, nn1_b1, nn1_W2, nn1_b2, root1, bias1,
           nn2_W1, nn2_b1, nn2_W2, nn2_b2, root2, bias2,
           fc1_W, fc1_b, fc2_W, fc2_b, fc3_W, fc3_b):
    src = edge_index[0]
    dst = edge_index[1]
    x_pad = jnp.pad(x, ((0, NPAD - N), (0, 0)))
    x5 = x_pad[:, :CS]
    xc = x_pad[:, CS:]
    ea_aug_t = jnp.concatenate(
        [edge_attr.T, jnp.ones((1, E), jnp.float32)], axis=0)

    xj1 = _sc_gather(x_pad, src, FEAT)
    msg1 = _tc_msg(ea_aug_t, xj1, nn1_W1, nn1_b1, nn1_W2, nn1_b2, CS, 32, 6400)
    agg1 = _sc_scatter_add(msg1, dst, jnp.zeros((NPAD, 32), jnp.float32), 32)
    h1 = _tc_h1(agg1, x5, root1, bias1)

    xj2 = _sc_gather(h1, src, 32)
    msg2 = _tc_msg(ea_aug_t, xj2, nn2_W1, nn2_b1, nn2_W2, nn2_b2, 32, 64, 3200)
    agg2 = _sc_scatter_add(msg2, dst, jnp.zeros((NPAD, 64), jnp.float32), 64)

    seg = jnp.concatenate(
        [node_to_subgraph, jnp.full((NPAD - N,), SUB, jnp.int32)]).reshape(
            NPAD // 1024, 1, 1024)
    sub_sums = _tc_pool1(agg2, h1, root2, bias2, xc, seg)
    out = _tc_final(sub_sums, subgraph_to_graph,
                    fc1_W, fc1_b, fc2_W, fc2_b, fc3_W, fc3_b)
    return out.reshape(-1)


# speedup vs baseline: 2.3506x; 1.0034x over previous
import functools

import jax
import jax.numpy as jnp
from jax import lax
from jax.experimental import pallas as pl
from jax.experimental.pallas import tpu as pltpu
import jax.experimental.pallas.tpu_sc as plsc

N = 10000
E = 160000
SUB = 1000
G = 64
FEAT = 16
CS = 5

NPAD = 10240
NC = 2
NS = 16
NW = NC * NS
EPT = E // NW
CH = 1000
NCHUNK = EPT // CH
RPT = NPAD // NS


def _elu(v):
    return jnp.where(v > 0, v, jnp.exp(v) - 1.0)


def _sc_gather(table, idx, d):
    mesh = plsc.VectorSubcoreMesh(core_axis_name="c", subcore_axis_name="s")

    @functools.partial(
        pl.kernel,
        out_type=jax.ShapeDtypeStruct((E, d), jnp.float32),
        mesh=mesh,
        scratch_types=[
            pltpu.VMEM((EPT,), jnp.int32),
            pltpu.VMEM((2, CH, d), jnp.float32),
            pltpu.SemaphoreType.DMA,
            pltpu.SemaphoreType.DMA,
            pltpu.SemaphoreType.DMA,
            pltpu.SemaphoreType.DMA,
        ],
        compiler_params=pltpu.CompilerParams(use_tc_tiling_on_sc=False),
    )
    def k(table_hbm, idx_hbm, out_hbm, idx_v, rows_v, g0, g1, s0, s1):
        wid = lax.axis_index("s") * NC + lax.axis_index("c")
        base = wid * EPT
        gsem = (g0, g1)
        ssem = (s0, s1)
        pltpu.sync_copy(idx_hbm.at[pl.ds(base, EPT)], idx_v)
        gathers = [pltpu.async_copy(
            table_hbm.at[idx_v.at[pl.ds(0, CH)]], rows_v.at[0], gsem[0])]
        stores = []
        for j in range(NCHUNK):
            b = j % 2
            gathers[j].wait()
            if j >= 1:
                stores[j - 1].wait()
            if j + 1 < NCHUNK:
                gathers.append(pltpu.async_copy(
                    table_hbm.at[idx_v.at[pl.ds((j + 1) * CH, CH)]],
                    rows_v.at[1 - b], gsem[1 - b]))
            stores.append(pltpu.async_copy(
                rows_v.at[b], out_hbm.at[pl.ds(base + j * CH, CH)], ssem[b]))
        stores[NCHUNK - 1].wait()

    return k(table, idx)


def _sc_scatter_add(msg, dst, zeros, d):
    nbuf = 2 if d <= 32 else 1
    mesh = plsc.VectorSubcoreMesh(core_axis_name="c", subcore_axis_name="s")

    @functools.partial(
        pl.kernel,
        out_type=jax.ShapeDtypeStruct((NC, NPAD, d), jnp.float32),
        mesh=mesh,
        scratch_types=[
            pltpu.VMEM((nbuf, CH), jnp.int32),
            pltpu.VMEM((nbuf, CH, d), jnp.float32),
            pltpu.VMEM_SHARED((NPAD, d), jnp.float32),
            pltpu.SemaphoreType.DMA,
            pltpu.SemaphoreType.DMA,
            pltpu.SemaphoreType.DMA,
        ],
        compiler_params=pltpu.CompilerParams(use_tc_tiling_on_sc=False),
    )
    def k(msg_hbm, dst_hbm, z_hbm, out_hbm, idx_v, rows_v, acc, isem, l0, l1):
        c = lax.axis_index("c")
        s = lax.axis_index("s")
        rbase = s * RPT
        lsem = (l0, l1)
        zcp = pltpu.async_copy(z_hbm.at[pl.ds(rbase, RPT)],
                               acc.at[pl.ds(rbase, RPT)], isem)
        base = (s * NC + c) * EPT
        iloads = [pltpu.async_copy(
            dst_hbm.at[pl.ds(base, CH)], idx_v.at[0], lsem[0])]
        loads = [pltpu.async_copy(
            msg_hbm.at[pl.ds(base, CH)], rows_v.at[0], lsem[0])]
        zcp.wait()
        plsc.subcore_barrier()
        for j in range(NCHUNK):
            b = (j % 2) if nbuf == 2 else 0
            nb = 1 - b if nbuf == 2 else 0
            iloads[j].wait()
            loads[j].wait()
            if nbuf == 1:
                pltpu.sync_copy(rows_v.at[b], acc.at[idx_v.at[b]], add=True)
            if j + 1 < NCHUNK:
                iloads.append(pltpu.async_copy(
                    dst_hbm.at[pl.ds(base + (j + 1) * CH, CH)],
                    idx_v.at[nb], lsem[nb]))
                loads.append(pltpu.async_copy(
                    msg_hbm.at[pl.ds(base + (j + 1) * CH, CH)],
                    rows_v.at[nb], lsem[nb]))
            if nbuf == 2:
                pltpu.sync_copy(rows_v.at[b], acc.at[idx_v.at[b]], add=True)
        plsc.subcore_barrier()
        pltpu.sync_copy(acc.at[pl.ds(rbase, RPT)], out_hbm.at[c, pl.ds(rbase, RPT)])

    return k(msg, dst, zeros)


def _msg_body(m_in, m_out, eat_ref, xj_ref, w1t_ref, w2t_ref, b2t_ref, o_ref):
    h_t = jnp.maximum(jnp.dot(w1t_ref[...], eat_ref[...],
                              preferred_element_type=jnp.float32), 0.0)
    we_t = jnp.dot(w2t_ref[...], h_t.astype(jnp.bfloat16),
                   preferred_element_type=jnp.float32)
    xj_t = jnp.transpose(xj_ref[...])[:m_in, :]
    acc = jnp.dot(b2t_ref[...], xj_t, preferred_element_type=jnp.float32)
    for i in range(m_in):
        acc = acc + xj_t[i:i + 1, :] * we_t[i * m_out:(i + 1) * m_out, :]
    o_ref[...] = jnp.transpose(acc)


def _tc_msg(ea_aug_t, xj, w1, b1, w2, b2, m_in, m_out, blk):
    grid = E // blk
    w1t = jnp.concatenate([w1.T, b1.reshape(-1, 1)], axis=1)
    w2t = w2.T.astype(jnp.bfloat16)
    b2t = b2.reshape(m_in, m_out).T
    return pl.pallas_call(
        functools.partial(_msg_body, m_in, m_out),
        grid=(grid,),
        in_specs=[
            pl.BlockSpec((6, blk), lambda i: (0, i)),
            pl.BlockSpec((blk, xj.shape[1]), lambda i: (i, 0)),
            pl.BlockSpec(w1t.shape, lambda i: (0, 0)),
            pl.BlockSpec(w2t.shape, lambda i: (0, 0)),
            pl.BlockSpec(b2t.shape, lambda i: (0, 0)),
        ],
        out_specs=pl.BlockSpec((blk, m_out), lambda i: (i, 0)),
        out_shape=jax.ShapeDtypeStruct((E, m_out), jnp.float32),
    )(ea_aug_t, xj, w1t, w2t, b2t)


def _h1_body(agg_ref, x5_ref, root_ref, bias_ref, o_ref):
    a = agg_ref[0] + agg_ref[1]
    v = a + jnp.dot(x5_ref[...], root_ref[...],
                    preferred_element_type=jnp.float32) + bias_ref[...]
    o_ref[...] = _elu(v)


def _tc_h1(agg, x5, root, bias):
    blk = 1024
    grid = NPAD // blk
    return pl.pallas_call(
        _h1_body,
        grid=(grid,),
        in_specs=[
            pl.BlockSpec((NC, blk, 32), lambda i: (0, i, 0)),
            pl.BlockSpec((blk, 5), lambda i: (i, 0)),
            pl.BlockSpec((5, 32), lambda i: (0, 0)),
            pl.BlockSpec((1, 32), lambda i: (0, 0)),
        ],
        out_specs=pl.BlockSpec((blk, 32), lambda i: (i, 0)),
        out_shape=jax.ShapeDtypeStruct((NPAD, 32), jnp.float32),
    )(agg, x5, root, bias.reshape(1, 32))


def _pool1_body(agg_ref, h1_ref, root_ref, bias_ref, xc_ref, seg_ref,
                s2g_ref, w1_ref, b1_ref, w2_ref, b2_ref, w3_ref, b3_ref,
                ss_ref, o_ref):
    i = pl.program_id(0)
    a = agg_ref[0] + agg_ref[1]
    h2 = _elu(a + jnp.dot(h1_ref[...], root_ref[...],
                          preferred_element_type=jnp.float32) + bias_ref[...])
    bn = h2.shape[0]
    ones = jnp.ones((bn, 1), jnp.float32)
    feat = jnp.concatenate([h2, xc_ref[...], ones], axis=1)
    seg = seg_ref[0]
    iota = lax.broadcasted_iota(jnp.int32, (SUB, bn), 0)
    oh = (iota == seg).astype(jnp.float32)

    @pl.when(i == 0)
    def _():
        ss_ref[...] = jnp.zeros_like(ss_ref)

    ss_ref[...] += jnp.dot(oh, feat, preferred_element_type=jnp.float32)

    @pl.when(i == pl.num_programs(0) - 1)
    def _():
        ss = ss_ref[...]
        cnt1 = ss[:, 75:76]
        mean1 = ss / jnp.maximum(cnt1, 1.0)
        s2g = s2g_ref[...]
        iota2 = lax.broadcasted_iota(jnp.int32, (G, SUB), 0)
        oh2 = (iota2 == s2g).astype(jnp.float32)
        gsum = jnp.dot(oh2, mean1, preferred_element_type=jnp.float32)
        cnt2 = jnp.sum(oh2, axis=1, keepdims=True)
        gmean = gsum / jnp.maximum(cnt2, 1.0)
        h = gmean[:, :75]
        h = _elu(jnp.dot(h, w1_ref[...],
                         preferred_element_type=jnp.float32) + b1_ref[...])
        h = _elu(jnp.dot(h, w2_ref[...],
                         preferred_element_type=jnp.float32) + b2_ref[...])
        o_ref[...] = jnp.dot(h, w3_ref[...],
                             preferred_element_type=jnp.float32) + b3_ref[...]


def _tc_pool_head(agg2, h1, root2, bias2, xc, seg3d, s2g,
                  fc1_W, fc1_b, fc2_W, fc2_b, fc3_W, fc3_b):
    blk = 1024
    grid = NPAD // blk
    _, out = pl.pallas_call(
        _pool1_body,
        grid=(grid,),
        in_specs=[
            pl.BlockSpec((NC, blk, 64), lambda i: (0, i, 0)),
            pl.BlockSpec((blk, 32), lambda i: (i, 0)),
            pl.BlockSpec((32, 64), lambda i: (0, 0)),
            pl.BlockSpec((1, 64), lambda i: (0, 0)),
            pl.BlockSpec((blk, 11), lambda i: (i, 0)),
            pl.BlockSpec((1, 1, blk), lambda i: (i, 0, 0)),
            pl.BlockSpec((1, SUB), lambda i: (0, 0)),
            pl.BlockSpec((75, 32), lambda i: (0, 0)),
            pl.BlockSpec((1, 32), lambda i: (0, 0)),
            pl.BlockSpec((32, 16), lambda i: (0, 0)),
            pl.BlockSpec((1, 16), lambda i: (0, 0)),
            pl.BlockSpec((16, 1), lambda i: (0, 0)),
            pl.BlockSpec((1, 1), lambda i: (0, 0)),
        ],
        out_specs=[
            pl.BlockSpec((SUB, 76), lambda i: (0, 0)),
            pl.BlockSpec((G, 1), lambda i: (0, 0)),
        ],
        out_shape=[
            jax.ShapeDtypeStruct((SUB, 76), jnp.float32),
            jax.ShapeDtypeStruct((G, 1), jnp.float32),
        ],
    )(agg2, h1, root2, bias2.reshape(1, 64), xc, seg3d,
      s2g.reshape(1, SUB), fc1_W, fc1_b.reshape(1, 32),
      fc2_W, fc2_b.reshape(1, 16), fc3_W, fc3_b.reshape(1, 1))
    return out


def _final_body(ss_ref, s2g_ref, w1_ref, b1_ref, w2_ref, b2_ref, w3_ref, b3_ref,
                o_ref):
    ss = ss_ref[...]
    cnt1 = ss[:, 75:76]
    mean1 = ss / jnp.maximum(cnt1, 1.0)
    s2g = s2g_ref[...]
    iota = lax.broadcasted_iota(jnp.int32, (G, SUB), 0)
    oh = (iota == s2g).astype(jnp.float32)
    gsum = jnp.dot(oh, mean1, preferred_element_type=jnp.float32)
    cnt2 = jnp.sum(oh, axis=1, keepdims=True)
    gmean = gsum / jnp.maximum(cnt2, 1.0)
    h = gmean[:, :75]
    h = _elu(jnp.dot(h, w1_ref[...], preferred_element_type=jnp.float32) + b1_ref[...])
    h = _elu(jnp.dot(h, w2_ref[...], preferred_element_type=jnp.float32) + b2_ref[...])
    o_ref[...] = jnp.dot(h, w3_ref[...], preferred_element_type=jnp.float32) + b3_ref[...]


def _tc_final(sub_sums, s2g, fc1_W, fc1_b, fc2_W, fc2_b, fc3_W, fc3_b):
    return pl.pallas_call(
        _final_body,
        out_shape=jax.ShapeDtypeStruct((G, 1), jnp.float32),
    )(sub_sums, s2g.reshape(1, SUB), fc1_W, fc1_b.reshape(1, 32),
      fc2_W, fc2_b.reshape(1, 16), fc3_W, fc3_b.reshape(1, 1))


def kernel(x, edge_index, edge_attr, node_to_subgraph, subgraph_to_graph,
           nn1_W1, nn1_b1, nn1_W2, nn1_b2, root1, bias1,
           nn2_W1, nn2_b1, nn2_W2, nn2_b2, root2, bias2,
           fc1_W, fc1_b, fc2_W, fc2_b, fc3_W, fc3_b):
    src = edge_index[0]
    dst = edge_index[1]
    x_pad = jnp.pad(x, ((0, NPAD - N), (0, 0)))
    x5 = x_pad[:, :CS]
    xc = x_pad[:, CS:]
    ea_aug_t = jnp.concatenate(
        [edge_attr.T, jnp.ones((1, E), jnp.float32)], axis=0)

    xj1 = _sc_gather(x_pad, src, FEAT)
    msg1 = _tc_msg(ea_aug_t, xj1, nn1_W1, nn1_b1, nn1_W2, nn1_b2, CS, 32, 6400)
    agg1 = _sc_scatter_add(msg1, dst, jnp.zeros((NPAD, 32), jnp.float32), 32)
    h1 = _tc_h1(agg1, x5, root1, bias1)

    xj2 = _sc_gather(h1, src, 32)
    msg2 = _tc_msg(ea_aug_t, xj2, nn2_W1, nn2_b1, nn2_W2, nn2_b2, 32, 64, 3200)
    agg2 = _sc_scatter_add(msg2, dst, jnp.zeros((NPAD, 64), jnp.float32), 64)

    seg = jnp.concatenate(
        [node_to_subgraph, jnp.full((NPAD - N,), SUB, jnp.int32)]).reshape(
            NPAD // 1024, 1, 1024)
    out = _tc_pool_head(agg2, h1, root2, bias2, xc, seg, subgraph_to_graph,
                        fc1_W, fc1_b, fc2_W, fc2_b, fc3_W, fc3_b)
    return out.reshape(-1)
